# Initial kernel scaffold; baseline (speedup 1.0000x reference)
import jax
import jax.numpy as jnp
from jax.experimental import pallas as pl


def kernel(x, W1, b1, W2, b2):
    raise NotImplementedError("write your pallas kernel here")


# speedup vs baseline: 1.8506x; 1.8506x over previous
import functools

import jax
import jax.numpy as jnp
from jax.experimental import pallas as pl
from jax.experimental.pallas import tpu as pltpu

INPUT_DIM = 768
HIDDEN = INPUT_DIM // 2
NUM_EXPERTS = 64
N_TOKENS = 32768
BT = 1024


def _gate_kernel(x_ref, w1_ref, b1_ref, w2_ref, b2_ref, w_out_ref, i_out_ref):
    h = jnp.dot(x_ref[:], w1_ref[:], preferred_element_type=jnp.float32)
    h = jnp.maximum(h + b1_ref[:], 0.0)
    logits = jnp.dot(h, w2_ref[:], preferred_element_type=jnp.float32)
    logits = logits + b2_ref[:]

    expert_ids = jax.lax.broadcasted_iota(jnp.int32, logits.shape, 1)
    m1 = jnp.max(logits, axis=-1, keepdims=True)
    i1 = jnp.min(jnp.where(logits == m1, expert_ids, NUM_EXPERTS), axis=-1,
                 keepdims=True)
    masked = jnp.where(expert_ids == i1, -jnp.inf, logits)
    m2 = jnp.max(masked, axis=-1, keepdims=True)
    i2 = jnp.min(jnp.where(masked == m2, expert_ids, NUM_EXPERTS), axis=-1,
                 keepdims=True)

    e = jnp.exp(m2 - m1)
    inv = 1.0 / (1.0 + e)
    pair = jax.lax.broadcasted_iota(jnp.int32, (x_ref.shape[0], 2), 1)
    w_out_ref[:] = jnp.where(pair == 0, inv, e * inv)
    i_out_ref[:] = jnp.where(pair == 0, i1, i2)


@jax.jit
def kernel(x, W1, b1, W2, b2):
    n = x.shape[0]
    grid = (n // BT,)
    out = pl.pallas_call(
        _gate_kernel,
        grid=grid,
        in_specs=[
            pl.BlockSpec((BT, INPUT_DIM), lambda i: (i, 0)),
            pl.BlockSpec((INPUT_DIM, HIDDEN), lambda i: (0, 0)),
            pl.BlockSpec((1, HIDDEN), lambda i: (0, 0)),
            pl.BlockSpec((HIDDEN, NUM_EXPERTS), lambda i: (0, 0)),
            pl.BlockSpec((1, NUM_EXPERTS), lambda i: (0, 0)),
        ],
        out_specs=[
            pl.BlockSpec((BT, 2), lambda i: (i, 0)),
            pl.BlockSpec((BT, 2), lambda i: (i, 0)),
        ],
        out_shape=[
            jax.ShapeDtypeStruct((n, 2), jnp.float32),
            jax.ShapeDtypeStruct((n, 2), jnp.int32),
        ],
        compiler_params=pltpu.CompilerParams(
            dimension_semantics=("arbitrary",),
        ),
    )(x, W1, b1.reshape(1, HIDDEN), W2, b2.reshape(1, NUM_EXPERTS))
    return (out[0], out[1])


# speedup vs baseline: 1.9465x; 1.0518x over previous
import functools

import jax
import jax.numpy as jnp
from jax.experimental import pallas as pl
from jax.experimental.pallas import tpu as pltpu

INPUT_DIM = 768
HIDDEN = INPUT_DIM // 2
NUM_EXPERTS = 64
N_TOKENS = 32768
BT = 1024


def _gate_kernel(x_ref, w1_ref, b1_ref, w2_ref, b2_ref, w_out_ref, i_out_ref):
    h = jnp.dot(x_ref[:], w1_ref[:], preferred_element_type=jnp.float32)
    h = jnp.maximum(h + b1_ref[:], 0.0)
    logits = jnp.dot(h, w2_ref[:], preferred_element_type=jnp.float32)
    logits = logits + b2_ref[:]

    lane_f = jax.lax.broadcasted_iota(jnp.int32, logits.shape, 1).astype(
        jnp.float32)
    m1 = jnp.max(logits, axis=-1, keepdims=True)
    i1f = jnp.min(jnp.where(logits == m1, lane_f, float(NUM_EXPERTS)),
                  axis=-1, keepdims=True)
    masked = jnp.where(lane_f == i1f, -jnp.inf, logits)
    m2 = jnp.max(masked, axis=-1, keepdims=True)
    i2f = jnp.min(jnp.where(masked == m2, lane_f, float(NUM_EXPERTS)),
                  axis=-1, keepdims=True)

    e = jnp.exp(m2 - m1)
    inv = 1.0 / (1.0 + e)
    pair = jax.lax.broadcasted_iota(jnp.int32, (x_ref.shape[0], 2), 1)
    w_out_ref[:] = jnp.where(pair == 0, inv, e * inv)
    i_out_ref[:] = jnp.where(pair == 0, i1f, i2f).astype(jnp.int32)


@jax.jit
def kernel(x, W1, b1, W2, b2):
    n = x.shape[0]
    grid = (n // BT,)
    out = pl.pallas_call(
        _gate_kernel,
        grid=grid,
        in_specs=[
            pl.BlockSpec((BT, INPUT_DIM), lambda i: (i, 0)),
            pl.BlockSpec((INPUT_DIM, HIDDEN), lambda i: (0, 0)),
            pl.BlockSpec((1, HIDDEN), lambda i: (0, 0)),
            pl.BlockSpec((HIDDEN, NUM_EXPERTS), lambda i: (0, 0)),
            pl.BlockSpec((1, NUM_EXPERTS), lambda i: (0, 0)),
        ],
        out_specs=[
            pl.BlockSpec((BT, 2), lambda i: (i, 0)),
            pl.BlockSpec((BT, 2), lambda i: (i, 0)),
        ],
        out_shape=[
            jax.ShapeDtypeStruct((n, 2), jnp.float32),
            jax.ShapeDtypeStruct((n, 2), jnp.int32),
        ],
        compiler_params=pltpu.CompilerParams(
            dimension_semantics=("arbitrary",),
        ),
    )(x, W1, b1.reshape(1, HIDDEN), W2, b2.reshape(1, NUM_EXPERTS))
    return (out[0], out[1])


# speedup vs baseline: 2.1809x; 1.1205x over previous
import functools

import jax
import jax.numpy as jnp
from jax.experimental import pallas as pl
from jax.experimental.pallas import tpu as pltpu

INPUT_DIM = 768
HIDDEN = INPUT_DIM // 2
NUM_EXPERTS = 64
N_TOKENS = 32768
BT = 2048


def _gate_kernel(x_ref, w1_ref, b1_ref, w2_ref, b2_ref, w_out_ref, i_out_ref):
    h = jnp.dot(x_ref[:], w1_ref[:], preferred_element_type=jnp.float32)
    h = jnp.maximum(h + b1_ref[:], 0.0)
    logits = jnp.dot(h, w2_ref[:], preferred_element_type=jnp.float32)
    logits = logits + b2_ref[:]

    lane_f = jax.lax.broadcasted_iota(jnp.int32, logits.shape, 1).astype(
        jnp.float32)
    m1 = jnp.max(logits, axis=-1, keepdims=True)
    i1f = jnp.min(jnp.where(logits == m1, lane_f, float(NUM_EXPERTS)),
                  axis=-1, keepdims=True)
    masked = jnp.where(lane_f == i1f, -jnp.inf, logits)
    m2 = jnp.max(masked, axis=-1, keepdims=True)
    i2f = jnp.min(jnp.where(masked == m2, lane_f, float(NUM_EXPERTS)),
                  axis=-1, keepdims=True)

    e = jnp.exp(m2 - m1)
    inv = 1.0 / (1.0 + e)
    pair = jax.lax.broadcasted_iota(jnp.int32, (x_ref.shape[0], 2), 1)
    w_out_ref[:] = jnp.where(pair == 0, inv, e * inv)
    i_out_ref[:] = jnp.where(pair == 0, i1f, i2f).astype(jnp.int32)


@jax.jit
def kernel(x, W1, b1, W2, b2):
    n = x.shape[0]
    grid = (n // BT,)
    out = pl.pallas_call(
        _gate_kernel,
        grid=grid,
        in_specs=[
            pl.BlockSpec((BT, INPUT_DIM), lambda i: (i, 0)),
            pl.BlockSpec((INPUT_DIM, HIDDEN), lambda i: (0, 0)),
            pl.BlockSpec((1, HIDDEN), lambda i: (0, 0)),
            pl.BlockSpec((HIDDEN, NUM_EXPERTS), lambda i: (0, 0)),
            pl.BlockSpec((1, NUM_EXPERTS), lambda i: (0, 0)),
        ],
        out_specs=[
            pl.BlockSpec((BT, 2), lambda i: (i, 0)),
            pl.BlockSpec((BT, 2), lambda i: (i, 0)),
        ],
        out_shape=[
            jax.ShapeDtypeStruct((n, 2), jnp.float32),
            jax.ShapeDtypeStruct((n, 2), jnp.int32),
        ],
        compiler_params=pltpu.CompilerParams(
            dimension_semantics=("arbitrary",),
        ),
    )(x, W1, b1.reshape(1, HIDDEN), W2, b2.reshape(1, NUM_EXPERTS))
    return (out[0], out[1])


# speedup vs baseline: 2.2563x; 1.0346x over previous
import functools

import jax
import jax.numpy as jnp
from jax.experimental import pallas as pl
from jax.experimental.pallas import tpu as pltpu

INPUT_DIM = 768
HIDDEN = INPUT_DIM // 2
NUM_EXPERTS = 64
N_TOKENS = 32768
BT = 4096


def _gate_kernel(x_ref, w1_ref, b1_ref, w2_ref, b2_ref, w_out_ref, i_out_ref):
    h = jnp.dot(x_ref[:], w1_ref[:], preferred_element_type=jnp.float32)
    h = jnp.maximum(h + b1_ref[:], 0.0)
    logits = jnp.dot(h, w2_ref[:], preferred_element_type=jnp.float32)
    logits = logits + b2_ref[:]

    lane_f = jax.lax.broadcasted_iota(jnp.int32, logits.shape, 1).astype(
        jnp.float32)
    m1 = jnp.max(logits, axis=-1, keepdims=True)
    i1f = jnp.min(jnp.where(logits == m1, lane_f, float(NUM_EXPERTS)),
                  axis=-1, keepdims=True)
    masked = jnp.where(lane_f == i1f, -jnp.inf, logits)
    m2 = jnp.max(masked, axis=-1, keepdims=True)
    i2f = jnp.min(jnp.where(masked == m2, lane_f, float(NUM_EXPERTS)),
                  axis=-1, keepdims=True)

    e = jnp.exp(m2 - m1)
    inv = 1.0 / (1.0 + e)
    pair = jax.lax.broadcasted_iota(jnp.int32, (x_ref.shape[0], 2), 1)
    w_out_ref[:] = jnp.where(pair == 0, inv, e * inv)
    i_out_ref[:] = jnp.where(pair == 0, i1f, i2f).astype(jnp.int32)


@jax.jit
def kernel(x, W1, b1, W2, b2):
    n = x.shape[0]
    grid = (n // BT,)
    out = pl.pallas_call(
        _gate_kernel,
        grid=grid,
        in_specs=[
            pl.BlockSpec((BT, INPUT_DIM), lambda i: (i, 0)),
            pl.BlockSpec((INPUT_DIM, HIDDEN), lambda i: (0, 0)),
            pl.BlockSpec((1, HIDDEN), lambda i: (0, 0)),
            pl.BlockSpec((HIDDEN, NUM_EXPERTS), lambda i: (0, 0)),
            pl.BlockSpec((1, NUM_EXPERTS), lambda i: (0, 0)),
        ],
        out_specs=[
            pl.BlockSpec((BT, 2), lambda i: (i, 0)),
            pl.BlockSpec((BT, 2), lambda i: (i, 0)),
        ],
        out_shape=[
            jax.ShapeDtypeStruct((n, 2), jnp.float32),
            jax.ShapeDtypeStruct((n, 2), jnp.int32),
        ],
        compiler_params=pltpu.CompilerParams(
            dimension_semantics=("arbitrary",),
        ),
    )(x, W1, b1.reshape(1, HIDDEN), W2, b2.reshape(1, NUM_EXPERTS))
    return (out[0], out[1])


# speedup vs baseline: 2.2615x; 1.0023x over previous
import functools

import jax
import jax.numpy as jnp
from jax.experimental import pallas as pl
from jax.experimental.pallas import tpu as pltpu

INPUT_DIM = 768
HIDDEN = INPUT_DIM // 2
NUM_EXPERTS = 64
N_TOKENS = 32768
BT = 4096


def _gate_kernel(x_ref, w1_ref, b1_ref, w2_ref, b2_ref, w_out_ref, i_out_ref):
    h = jnp.dot(x_ref[:], w1_ref[:], preferred_element_type=jnp.float32)
    h = jnp.maximum(h + b1_ref[:], 0.0)
    logits = jnp.dot(h, w2_ref[:], preferred_element_type=jnp.float32)
    logits = logits + b2_ref[:]

    lane_f = jax.lax.broadcasted_iota(jnp.int32, logits.shape, 1).astype(
        jnp.float32)
    m1 = jnp.max(logits, axis=-1, keepdims=True)
    i1f = jnp.min(jnp.where(logits == m1, lane_f, float(NUM_EXPERTS)),
                  axis=-1, keepdims=True)
    masked = jnp.where(lane_f == i1f, -jnp.inf, logits)
    m2 = jnp.max(masked, axis=-1, keepdims=True)
    i2f = jnp.min(jnp.where(masked == m2, lane_f, float(NUM_EXPERTS)),
                  axis=-1, keepdims=True)

    e = jnp.exp(m2 - m1)
    inv = 1.0 / (1.0 + e)
    pair = jax.lax.broadcasted_iota(jnp.int32, (x_ref.shape[0], 2), 1)
    w_out_ref[:] = jnp.where(pair == 0, inv, e * inv)
    i_out_ref[:] = jnp.where(pair == 0, i1f, i2f).astype(jnp.int32)


@jax.jit
def kernel(x, W1, b1, W2, b2):
    n = x.shape[0]
    grid = (n // BT,)
    out = pl.pallas_call(
        _gate_kernel,
        grid=grid,
        in_specs=[
            pl.BlockSpec((BT, INPUT_DIM), lambda i: (i, 0)),
            pl.BlockSpec((INPUT_DIM, HIDDEN), lambda i: (0, 0)),
            pl.BlockSpec((1, HIDDEN), lambda i: (0, 0)),
            pl.BlockSpec((HIDDEN, NUM_EXPERTS), lambda i: (0, 0)),
            pl.BlockSpec((1, NUM_EXPERTS), lambda i: (0, 0)),
        ],
        out_specs=[
            pl.BlockSpec((BT, 2), lambda i: (i, 0)),
            pl.BlockSpec((BT, 2), lambda i: (i, 0)),
        ],
        out_shape=[
            jax.ShapeDtypeStruct((n, 2), jnp.float32),
            jax.ShapeDtypeStruct((n, 2), jnp.int32),
        ],
        compiler_params=pltpu.CompilerParams(
            dimension_semantics=("parallel",),
        ),
    )(x, W1, b1.reshape(1, HIDDEN), W2, b2.reshape(1, NUM_EXPERTS))
    return (out[0], out[1])
